# Initial kernel scaffold; baseline (speedup 1.0000x reference)
import jax
import jax.numpy as jnp
from jax.experimental import pallas as pl


def kernel(pos_u, pos_v, neg_v, u_weight, v_weight):
    raise NotImplementedError("write your pallas kernel here")


# speedup vs baseline: 1.7292x; 1.7292x over previous
import functools

import jax
import jax.numpy as jnp
from jax import lax
from jax.experimental import pallas as pl
from jax.experimental.pallas import tpu as pltpu
from jax.experimental.pallas import tpu_sc as plsc

NC = 2
NSUB = 16
NW = NC * NSUB
L = 16

B = 16384
D = 64
NNEG = 5
BPW = B // NW
CH = 128
NCHUNK = BPW // CH
GRP = CH // L
DV = D // L


def _sc_body(pos_u, pos_v, neg_v, u_w, v_w, out,
             uidx_v, vidx_v, nidx_v, urows_v, vrows_v, nrows_v, scores_v, sem):
    wid = lax.axis_index("s") * NC + lax.axis_index("c")
    base = wid * BPW
    pltpu.sync_copy(pos_u.at[pl.ds(base, BPW)], uidx_v)
    pltpu.sync_copy(pos_v.at[pl.ds(base, BPW)], vidx_v)
    pltpu.sync_copy(neg_v.at[pl.ds(base * NNEG, BPW * NNEG)], nidx_v)

    lane = lax.iota(jnp.int32, L)

    def chunk_body(c, _):
        cps = [
            pltpu.async_copy(u_w.at[uidx_v.at[pl.ds(c * CH, CH)]], urows_v, sem),
            pltpu.async_copy(v_w.at[vidx_v.at[pl.ds(c * CH, CH)]], vrows_v, sem),
        ]
        for q in range(NNEG):
            cps.append(pltpu.async_copy(
                v_w.at[nidx_v.at[pl.ds(c * (CH * NNEG) + q * CH, CH)]],
                nrows_v.at[pl.ds(q * CH, CH)], sem))
        for cp in cps:
            cp.wait()

        def group_body(g, _):
            accs = [jnp.zeros((L,), jnp.float32) for _ in range(1 + NNEG)]
            for j in range(L):
                e = g * L + j
                us = [urows_v[e, pl.ds(k * L, L)] for k in range(DV)]
                vs = [vrows_v[e, pl.ds(k * L, L)] for k in range(DV)]
                s = jnp.sum(sum(u * v for u, v in zip(us, vs)))
                accs[0] = jnp.where(lane == j, s, accs[0])
                for q in range(NNEG):
                    ns = [nrows_v[q * CH + e, pl.ds(k * L, L)] for k in range(DV)]
                    s = jnp.sum(sum(u * n for u, n in zip(us, ns)))
                    accs[1 + q] = jnp.where(lane == j, s, accs[1 + q])
            for r in range(1 + NNEG):
                scores_v[r, pl.ds(c * CH + g * L, L)] = accs[r]
            return 0

        lax.fori_loop(0, GRP, group_body, 0)
        return 0

    lax.fori_loop(0, NCHUNK, chunk_body, 0)
    pltpu.sync_copy(scores_v, out.at[:, pl.ds(base, BPW)])


@jax.jit
def _sc_scores(pos_u, pos_v, neg_v_flat, u_w, v_w):
    mesh = plsc.VectorSubcoreMesh(core_axis_name="c", subcore_axis_name="s")
    return pl.kernel(
        _sc_body,
        out_type=jax.ShapeDtypeStruct((1 + NNEG, B), jnp.float32),
        mesh=mesh,
        compiler_params=pltpu.CompilerParams(
            needs_layout_passes=False, use_tc_tiling_on_sc=False),
        scratch_types=[
            pltpu.VMEM((BPW,), jnp.int32),
            pltpu.VMEM((BPW,), jnp.int32),
            pltpu.VMEM((BPW * NNEG,), jnp.int32),
            pltpu.VMEM((CH, D), jnp.float32),
            pltpu.VMEM((CH, D), jnp.float32),
            pltpu.VMEM((CH * NNEG, D), jnp.float32),
            pltpu.VMEM((1 + NNEG, BPW), jnp.float32),
            pltpu.SemaphoreType.DMA,
        ],
    )(pos_u, pos_v, neg_v_flat, u_w, v_w)


def _loss_body(scores_ref, out_ref):
    s = scores_ref[...]
    row = lax.broadcasted_iota(jnp.int32, s.shape, 0)
    x = jnp.where(row == 0, s, -s)
    ls = jax.nn.log_sigmoid(x)
    out_ref[...] = jnp.full((1, 1), -jnp.sum(ls) / B, jnp.float32)


@jax.jit
def _loss(scores):
    out = pl.pallas_call(
        _loss_body,
        out_shape=jax.ShapeDtypeStruct((1, 1), jnp.float32),
    )(scores)
    return out[0, 0]


def kernel(pos_u, pos_v, neg_v, u_weight, v_weight):
    scores = _sc_scores(pos_u, pos_v, neg_v.reshape(-1), u_weight, v_weight)
    return _loss(scores)
